# initial kernel scaffold (unmeasured)
import jax
import jax.numpy as jnp
from jax import lax
from jax.experimental import pallas as pl
from jax.experimental.pallas import tpu as pltpu

N_DEV = 4
B_LOC = 2
SQ = 128
SKV = 128
HQ = 16
HG = HQ // N_DEV
DH = 64
D_MODEL = 512
DG = HG * DH
BLK = 64


def kernel(x, Wq, K_ext, V_ext, Wo):
    my = lax.axis_index("i")
    k_loc = lax.dynamic_slice_in_dim(K_ext, my * B_LOC, B_LOC, axis=0)
    v_loc = lax.dynamic_slice_in_dim(V_ext, my * B_LOC, B_LOC, axis=0)
    k_t = jnp.transpose(k_loc, (0, 2, 1, 3)).reshape(B_LOC * HQ, SKV, DH)
    v_t = jnp.transpose(v_loc, (0, 2, 1, 3)).reshape(B_LOC * HQ, SKV, DH)

    def body(x_ref, k_ref, v_ref, out_ref,
             wq_buf, wo_buf, ctx_ref,
             wq_send, wq_recv, wo_send, wo_recv):
        my_pos = lax.axis_index("i")
        left = (my_pos + N_DEV - 1) % N_DEV
        right = (my_pos + 1) % N_DEV

        barrier_sem = pltpu.get_barrier_semaphore()
        for nbr in (left, right):
            pl.semaphore_signal(
                barrier_sem, inc=1,
                device_id=(nbr,), device_id_type=pl.DeviceIdType.MESH,
            )
        pl.semaphore_wait(barrier_sem, 2)

        xf = x_ref[...].reshape(B_LOC * SQ, D_MODEL)

        qb = lax.broadcasted_iota(jnp.int32, (SQ, SKV), 0) // BLK
        kb = lax.broadcasted_iota(jnp.int32, (SQ, SKV), 1) // BLK
        mask = kb <= qb

        def compute(slot):
            origin = (my_pos + N_DEV - slot) % N_DEV
            q = jnp.dot(xf, wq_buf[slot], preferred_element_type=jnp.float32)
            for b in range(B_LOC):
                for hl in range(HG):
                    qh = q[b * SQ:(b + 1) * SQ, hl * DH:(hl + 1) * DH]
                    idx = b * HQ + origin * HG + hl
                    kh = k_ref[pl.ds(idx, 1), :, :].reshape(SKV, DH)
                    vh = v_ref[pl.ds(idx, 1), :, :].reshape(SKV, DH)
                    s = lax.dot_general(
                        qh, kh, (((1,), (1,)), ((), ())),
                        preferred_element_type=jnp.float32,
                    ) * 0.125
                    s = jnp.where(mask, s, -1e9)
                    m = jnp.max(s, axis=1, keepdims=True)
                    w = jnp.exp(s - m)
                    w = w / jnp.sum(w, axis=1, keepdims=True)
                    ctx_ref[b * SQ:(b + 1) * SQ, hl * DH:(hl + 1) * DH] = (
                        jnp.dot(w, vh, preferred_element_type=jnp.float32)
                    )
            partial = jnp.dot(
                ctx_ref[...], wo_buf[slot], preferred_element_type=jnp.float32
            ).reshape(B_LOC, SQ, D_MODEL)
            if slot == 0:
                out_ref[...] = partial
            else:
                out_ref[...] = out_ref[...] + partial

        def hop(h):
            rq = pltpu.make_async_remote_copy(
                src_ref=wq_buf.at[h], dst_ref=wq_buf.at[h + 1],
                send_sem=wq_send.at[h], recv_sem=wq_recv.at[h],
                device_id=(right,), device_id_type=pl.DeviceIdType.MESH,
            )
            ro = pltpu.make_async_remote_copy(
                src_ref=wo_buf.at[h], dst_ref=wo_buf.at[h + 1],
                send_sem=wo_send.at[h], recv_sem=wo_recv.at[h],
                device_id=(right,), device_id_type=pl.DeviceIdType.MESH,
            )
            rq.start()
            ro.start()
            return rq, ro

        wq_buf[0] = Wq[...] if False else wq_buf[0]

        return compute, hop

    def body(x_ref, wq_ref, k_ref, v_ref, wo_ref, out_ref,
             wq_buf, wo_buf, ctx_ref,
             wq_send, wq_recv, wo_send, wo_recv):
        my_pos = lax.axis_index("i")
        left = (my_pos + N_DEV - 1) % N_DEV
        right = (my_pos + 1) % N_DEV

        barrier_sem = pltpu.get_barrier_semaphore()
        for nbr in (left, right):
            pl.semaphore_signal(
                barrier_sem, inc=1,
                device_id=(nbr,), device_id_type=pl.DeviceIdType.MESH,
            )
        pl.semaphore_wait(barrier_sem, 2)

        xf = x_ref[...].reshape(B_LOC * SQ, D_MODEL)
        qb = lax.broadcasted_iota(jnp.int32, (SQ, SKV), 0) // BLK
        kb = lax.broadcasted_iota(jnp.int32, (SQ, SKV), 1) // BLK
        mask = kb <= qb

        def compute(slot):
            origin = (my_pos + N_DEV - slot) % N_DEV
            q = jnp.dot(xf, wq_buf[slot], preferred_element_type=jnp.float32)
            for b in range(B_LOC):
                for hl in range(HG):
                    qh = q[b * SQ:(b + 1) * SQ, hl * DH:(hl + 1) * DH]
                    idx = b * HQ + origin * HG + hl
                    kh = k_ref[pl.ds(idx, 1), :, :].reshape(SKV, DH)
                    vh = v_ref[pl.ds(idx, 1), :, :].reshape(SKV, DH)
                    s = lax.dot_general(
                        qh, kh, (((1,), (1,)), ((), ())),
                        preferred_element_type=jnp.float32,
                    ) * 0.125
                    s = jnp.where(mask, s, -1e9)
                    m = jnp.max(s, axis=1, keepdims=True)
                    w = jnp.exp(s - m)
                    w = w / jnp.sum(w, axis=1, keepdims=True)
                    ctx_ref[b * SQ:(b + 1) * SQ, hl * DH:(hl + 1) * DH] = (
                        jnp.dot(w, vh, preferred_element_type=jnp.float32)
                    )
            partial = jnp.dot(
                ctx_ref[...], wo_buf[slot], preferred_element_type=jnp.float32
            ).reshape(B_LOC, SQ, D_MODEL)
            if slot == 0:
                out_ref[...] = partial
            else:
                out_ref[...] = out_ref[...] + partial

        def make_hop(h):
            rq = pltpu.make_async_remote_copy(
                src_ref=wq_buf.at[h], dst_ref=wq_buf.at[h + 1],
                send_sem=wq_send.at[h], recv_sem=wq_recv.at[h],
                device_id=(right,), device_id_type=pl.DeviceIdType.MESH,
            )
            ro = pltpu.make_async_remote_copy(
                src_ref=wo_buf.at[h], dst_ref=wo_buf.at[h + 1],
                send_sem=wo_send.at[h], recv_sem=wo_recv.at[h],
                device_id=(right,), device_id_type=pl.DeviceIdType.MESH,
            )
            rq.start()
            ro.start()
            return rq, ro

        wq_buf[0] = wq_ref[...]
        wo_buf[0] = wo_ref[...]

        rdmas = []
        rdmas.append(make_hop(0))
        compute(0)
        for h in range(1, N_DEV):
            rq, ro = rdmas[h - 1]
            rq.wait_recv()
            ro.wait_recv()
            if h < N_DEV - 1:
                rdmas.append(make_hop(h))
            compute(h)

        for rq, ro in rdmas:
            rq.wait_send()
            ro.wait_send()

    return pl.pallas_call(
        body,
        out_shape=jax.ShapeDtypeStruct((B_LOC, SQ, D_MODEL), jnp.float32),
        in_specs=[
            pl.BlockSpec(memory_space=pltpu.VMEM),
            pl.BlockSpec(memory_space=pltpu.VMEM),
            pl.BlockSpec(memory_space=pltpu.VMEM),
            pl.BlockSpec(memory_space=pltpu.VMEM),
            pl.BlockSpec(memory_space=pltpu.VMEM),
        ],
        out_specs=pl.BlockSpec(memory_space=pltpu.VMEM),
        scratch_shapes=[
            pltpu.VMEM((N_DEV, D_MODEL, DG), jnp.float32),
            pltpu.VMEM((N_DEV, DG, D_MODEL), jnp.float32),
            pltpu.VMEM((B_LOC * SQ, DG), jnp.float32),
            pltpu.SemaphoreType.DMA((N_DEV - 1,)),
            pltpu.SemaphoreType.DMA((N_DEV - 1,)),
            pltpu.SemaphoreType.DMA((N_DEV - 1,)),
            pltpu.SemaphoreType.DMA((N_DEV - 1,)),
        ],
        compiler_params=pltpu.CompilerParams(collective_id=0),
    )(x, Wq, k_t, v_t, Wo)


# baseline (device time: 48632 ns/iter reference)
import jax
import jax.numpy as jnp
from jax import lax
from jax.experimental import pallas as pl
from jax.experimental.pallas import tpu as pltpu

N_DEV = 4
B_LOC = 2
SQ = 128
SKV = 128
HQ = 16
HG = HQ // N_DEV
DH = 64
D_MODEL = 512
DG = HG * DH
BLK = 64


def kernel(x, Wq, K_ext, V_ext, Wo):
    my = lax.axis_index("i")
    k_loc = lax.dynamic_slice_in_dim(K_ext, my * B_LOC, B_LOC, axis=0)
    v_loc = lax.dynamic_slice_in_dim(V_ext, my * B_LOC, B_LOC, axis=0)
    k_t = jnp.transpose(k_loc, (0, 2, 1, 3)).reshape(B_LOC * HQ, SKV, DH)
    v_t = jnp.transpose(v_loc, (0, 2, 1, 3)).reshape(B_LOC * HQ, SKV, DH)

    def body(x_ref, wq_ref, k_ref, v_ref, wo_ref, out_ref,
             wq_buf, wo_buf, ctx_ref,
             wq_send, wq_recv, wo_send, wo_recv):
        my_pos = lax.axis_index("i")
        left = (my_pos + N_DEV - 1) % N_DEV
        right = (my_pos + 1) % N_DEV

        barrier_sem = pltpu.get_barrier_semaphore()
        for nbr in (left, right):
            pl.semaphore_signal(
                barrier_sem, inc=1,
                device_id=(nbr,), device_id_type=pl.DeviceIdType.MESH,
            )
        pl.semaphore_wait(barrier_sem, 2)

        xf = x_ref[...].reshape(B_LOC * SQ, D_MODEL)
        qb = lax.broadcasted_iota(jnp.int32, (SQ, SKV), 0) // BLK
        kb = lax.broadcasted_iota(jnp.int32, (SQ, SKV), 1) // BLK
        mask = kb <= qb

        def compute(slot):
            origin = (my_pos + N_DEV - slot) % N_DEV
            q = jnp.dot(xf, wq_buf[slot], preferred_element_type=jnp.float32)
            for b in range(B_LOC):
                for hl in range(HG):
                    qh = q[b * SQ:(b + 1) * SQ, hl * DH:(hl + 1) * DH]
                    idx = b * HQ + origin * HG + hl
                    kh = k_ref[pl.ds(idx, 1), :, :].reshape(SKV, DH)
                    vh = v_ref[pl.ds(idx, 1), :, :].reshape(SKV, DH)
                    s = lax.dot_general(
                        qh, kh, (((1,), (1,)), ((), ())),
                        preferred_element_type=jnp.float32,
                    ) * 0.125
                    s = jnp.where(mask, s, -1e9)
                    m = jnp.max(s, axis=1, keepdims=True)
                    w = jnp.exp(s - m)
                    w = w / jnp.sum(w, axis=1, keepdims=True)
                    ctx_ref[b * SQ:(b + 1) * SQ, hl * DH:(hl + 1) * DH] = (
                        jnp.dot(w, vh, preferred_element_type=jnp.float32)
                    )
            partial = jnp.dot(
                ctx_ref[...], wo_buf[slot], preferred_element_type=jnp.float32
            ).reshape(B_LOC, SQ, D_MODEL)
            if slot == 0:
                out_ref[...] = partial
            else:
                out_ref[...] = out_ref[...] + partial

        def make_hop(h):
            rq = pltpu.make_async_remote_copy(
                src_ref=wq_buf.at[h], dst_ref=wq_buf.at[h + 1],
                send_sem=wq_send.at[h], recv_sem=wq_recv.at[h],
                device_id=(right,), device_id_type=pl.DeviceIdType.MESH,
            )
            ro = pltpu.make_async_remote_copy(
                src_ref=wo_buf.at[h], dst_ref=wo_buf.at[h + 1],
                send_sem=wo_send.at[h], recv_sem=wo_recv.at[h],
                device_id=(right,), device_id_type=pl.DeviceIdType.MESH,
            )
            rq.start()
            ro.start()
            return rq, ro

        wq_buf[0] = wq_ref[...]
        wo_buf[0] = wo_ref[...]

        rdmas = []
        rdmas.append(make_hop(0))
        compute(0)
        for h in range(1, N_DEV):
            rq, ro = rdmas[h - 1]
            rq.wait_recv()
            ro.wait_recv()
            if h < N_DEV - 1:
                rdmas.append(make_hop(h))
            compute(h)

        for rq, ro in rdmas:
            rq.wait_send()
            ro.wait_send()

    return pl.pallas_call(
        body,
        out_shape=jax.ShapeDtypeStruct((B_LOC, SQ, D_MODEL), jnp.float32),
        in_specs=[
            pl.BlockSpec(memory_space=pltpu.VMEM),
            pl.BlockSpec(memory_space=pltpu.VMEM),
            pl.BlockSpec(memory_space=pltpu.VMEM),
            pl.BlockSpec(memory_space=pltpu.VMEM),
            pl.BlockSpec(memory_space=pltpu.VMEM),
        ],
        out_specs=pl.BlockSpec(memory_space=pltpu.VMEM),
        scratch_shapes=[
            pltpu.VMEM((N_DEV, D_MODEL, DG), jnp.float32),
            pltpu.VMEM((N_DEV, DG, D_MODEL), jnp.float32),
            pltpu.VMEM((B_LOC * SQ, DG), jnp.float32),
            pltpu.SemaphoreType.DMA((N_DEV - 1,)),
            pltpu.SemaphoreType.DMA((N_DEV - 1,)),
            pltpu.SemaphoreType.DMA((N_DEV - 1,)),
            pltpu.SemaphoreType.DMA((N_DEV - 1,)),
        ],
        compiler_params=pltpu.CompilerParams(collective_id=0),
    )(x, Wq, k_t, v_t, Wo)


# device time: 29722 ns/iter; 1.6362x vs baseline; 1.6362x over previous
import jax
import jax.numpy as jnp
from jax import lax
from jax.experimental import pallas as pl
from jax.experimental.pallas import tpu as pltpu

N_DEV = 4
B_LOC = 2
SQ = 128
SKV = 128
HQ = 16
HG = HQ // N_DEV
DH = 64
D_MODEL = 512
DG = HG * DH
BLK = 64
WQ_H = D_MODEL // 2
WO_H = DG // 2


def kernel(x, Wq, K_ext, V_ext, Wo):
    my = lax.axis_index("i")
    k_loc = lax.dynamic_slice_in_dim(K_ext, my * B_LOC, B_LOC, axis=0)
    v_loc = lax.dynamic_slice_in_dim(V_ext, my * B_LOC, B_LOC, axis=0)
    k_t = jnp.transpose(k_loc, (0, 2, 1, 3)).reshape(B_LOC * HQ, SKV, DH)
    v_t = jnp.transpose(v_loc, (0, 2, 1, 3)).reshape(B_LOC * HQ, SKV, DH)

    def body(x_ref, wq_ref, k_ref, v_ref, wo_ref, out_ref,
             wq_l, wq_r, wq_d, wo_l, wo_r, wo_d, ctx_ref,
             send_sems, recv_sems):
        my_pos = lax.axis_index("i")
        left = (my_pos + N_DEV - 1) % N_DEV
        right = (my_pos + 1) % N_DEV

        barrier_sem = pltpu.get_barrier_semaphore()
        for nbr in (left, right):
            pl.semaphore_signal(
                barrier_sem, inc=1,
                device_id=(nbr,), device_id_type=pl.DeviceIdType.MESH,
            )
        pl.semaphore_wait(barrier_sem, 2)

        def rdma(i, src, dst, dev):
            return pltpu.make_async_remote_copy(
                src_ref=src, dst_ref=dst,
                send_sem=send_sems.at[i], recv_sem=recv_sems.at[i],
                device_id=(dev,), device_id_type=pl.DeviceIdType.MESH,
            )

        d0 = rdma(0, wq_ref, wq_l, right)
        d2 = rdma(2, wq_ref, wq_r, left)
        d1 = rdma(1, wo_ref, wo_l, right)
        d3 = rdma(3, wo_ref, wo_r, left)
        d0.start()
        d2.start()
        d1.start()
        d3.start()

        xf = x_ref[...].reshape(B_LOC * SQ, D_MODEL)

        qb = lax.broadcasted_iota(jnp.int32, (SQ, SKV), 0) // BLK
        kb = lax.broadcasted_iota(jnp.int32, (SQ, SKV), 1) // BLK
        mask = kb <= qb

        def compute(wq_g, wo_g, origin, first=False):
            q = jnp.dot(xf, wq_g, preferred_element_type=jnp.float32)
            for b in range(B_LOC):
                for hl in range(HG):
                    qh = q[b * SQ:(b + 1) * SQ, hl * DH:(hl + 1) * DH]
                    idx = b * HQ + origin * HG + hl
                    kh = k_ref[pl.ds(idx, 1), :, :].reshape(SKV, DH)
                    vh = v_ref[pl.ds(idx, 1), :, :].reshape(SKV, DH)
                    s = lax.dot_general(
                        qh, kh, (((1,), (1,)), ((), ())),
                        preferred_element_type=jnp.float32,
                    ) * 0.125
                    s = jnp.where(mask, s, -1e9)
                    m = jnp.max(s, axis=1, keepdims=True)
                    w = jnp.exp(s - m)
                    w = w / jnp.sum(w, axis=1, keepdims=True)
                    ctx_ref[b * SQ:(b + 1) * SQ, hl * DH:(hl + 1) * DH] = (
                        jnp.dot(w, vh, preferred_element_type=jnp.float32)
                    )
            partial = jnp.dot(
                ctx_ref[...], wo_g, preferred_element_type=jnp.float32
            ).reshape(B_LOC, SQ, D_MODEL)
            if first:
                out_ref[...] = partial
            else:
                out_ref[...] = out_ref[...] + partial

        compute(wq_ref[...], wo_ref[...], my_pos, first=True)

        d0.wait_recv()
        d1.wait_recv()
        d4 = rdma(4, wq_l.at[pl.ds(0, WQ_H)], wq_d.at[pl.ds(0, WQ_H)], right)
        d5 = rdma(5, wo_l.at[pl.ds(0, WO_H)], wo_d.at[pl.ds(0, WO_H)], right)
        d4.start()
        d5.start()

        d2.wait_recv()
        d3.wait_recv()
        d6 = rdma(6, wq_r.at[pl.ds(WQ_H, WQ_H)], wq_d.at[pl.ds(WQ_H, WQ_H)], left)
        d7 = rdma(7, wo_r.at[pl.ds(WO_H, WO_H)], wo_d.at[pl.ds(WO_H, WO_H)], left)
        d6.start()
        d7.start()

        compute(wq_l[...], wo_l[...], left)
        compute(wq_r[...], wo_r[...], right)

        d4.wait_recv()
        d5.wait_recv()
        d6.wait_recv()
        d7.wait_recv()
        compute(wq_d[...], wo_d[...], (my_pos + 2) % N_DEV)

        for d in (d0, d1, d2, d3, d4, d5, d6, d7):
            d.wait_send()

    return pl.pallas_call(
        body,
        out_shape=jax.ShapeDtypeStruct((B_LOC, SQ, D_MODEL), jnp.float32),
        in_specs=[
            pl.BlockSpec(memory_space=pltpu.VMEM),
            pl.BlockSpec(memory_space=pltpu.VMEM),
            pl.BlockSpec(memory_space=pltpu.VMEM),
            pl.BlockSpec(memory_space=pltpu.VMEM),
            pl.BlockSpec(memory_space=pltpu.VMEM),
        ],
        out_specs=pl.BlockSpec(memory_space=pltpu.VMEM),
        scratch_shapes=[
            pltpu.VMEM((D_MODEL, DG), jnp.float32),
            pltpu.VMEM((D_MODEL, DG), jnp.float32),
            pltpu.VMEM((D_MODEL, DG), jnp.float32),
            pltpu.VMEM((DG, D_MODEL), jnp.float32),
            pltpu.VMEM((DG, D_MODEL), jnp.float32),
            pltpu.VMEM((DG, D_MODEL), jnp.float32),
            pltpu.VMEM((B_LOC * SQ, DG), jnp.float32),
            pltpu.SemaphoreType.DMA((8,)),
            pltpu.SemaphoreType.DMA((8,)),
        ],
        compiler_params=pltpu.CompilerParams(collective_id=0),
    )(x, Wq, k_t, v_t, Wo)
